# baseline (speedup 1.0000x reference)
import functools

import jax
import jax.numpy as jnp
from jax import lax
from jax.experimental import pallas as pl
from jax.experimental.pallas import tpu as pltpu

B = 1
S = 2048
D = 768
H = 12
HD = D // H
P = 32
K = 2
R = 16
M = S // R
PR = P * R
NEG = -1e30


def _sse_head_kernel(x_ref, wq_ref, bq_ref, wk_ref, bk_ref, wv_ref, bv_ref,
                     pe_ref, wo_ref, bo_ref, y_ref, concat_ref):
    h = pl.program_id(0)
    xh = x_ref[:, 0, 0, :]
    wq = wq_ref[0]
    wk = wk_ref[0]
    wv = wv_ref[0]
    pe = pe_ref[0]

    q = jnp.dot(xh, wq, preferred_element_type=jnp.float32) + bq_ref[0]
    k = jnp.dot(xh, wk, preferred_element_type=jnp.float32) + bk_ref[0]
    v = jnp.dot(xh, wv, preferred_element_type=jnp.float32) + bv_ref[0]

    logits = jax.lax.dot_general(pe, q, (((1,), (1,)), ((), ())),
                                 preferred_element_type=jnp.float32)
    iota_p = lax.broadcasted_iota(jnp.int32, (P, S), 0)
    v1 = jnp.max(logits, axis=0, keepdims=True)
    i1 = jnp.min(jnp.where(logits == v1, iota_p, P), axis=0, keepdims=True)
    hit1 = iota_p == i1
    l2 = jnp.where(hit1, NEG, logits)
    v2 = jnp.max(l2, axis=0, keepdims=True)
    i2 = jnp.min(jnp.where(l2 == v2, iota_p, P), axis=0, keepdims=True)
    hit2 = iota_p == i2
    e = jnp.exp(v2 - v1)
    g1 = 1.0 / (1.0 + e)
    g2 = e / (1.0 + e)

    addm = jnp.where(hit1 | hit2, 0.0, NEG)

    i1c = jnp.swapaxes(i1, 0, 1)
    i2c = jnp.swapaxes(i2, 0, 1)
    g1c = jnp.swapaxes(g1, 0, 1)
    g2c = jnp.swapaxes(g2, 0, 1)
    iota_sp = lax.broadcasted_iota(jnp.int32, (S, P), 1)
    w_sp2 = (jnp.where(iota_sp == i1c, g1c, 0.0) +
             jnp.where(iota_sp == i2c, g2c, 0.0))

    w4 = w_sp2.reshape(M, R, P)
    k3 = k.reshape(M, R, HD)
    v3 = v.reshape(M, R, HD)
    dn = (((0,), (0,)), ((1,), (1,)))
    st_k = jax.lax.dot_general(w4, k3, dn,
                               preferred_element_type=jnp.float32).reshape(PR, HD)
    st_v = jax.lax.dot_general(w4, v3, dn,
                               preferred_element_type=jnp.float32).reshape(PR, HD)

    q8 = q * (1.0 / 8.0)
    scores = jax.lax.dot_general(st_k, q8, (((1,), (1,)), ((), ())),
                                 preferred_element_type=jnp.float32)
    addm_b = jnp.broadcast_to(addm[None, :, :], (R, P, S)).reshape(PR, S)
    masked = scores + addm_b
    m = jnp.max(masked, axis=0, keepdims=True)
    ex = jnp.exp(masked - m)
    attn = ex * (1.0 / jnp.sum(ex, axis=0, keepdims=True))

    out_ht = jax.lax.dot_general(st_v.astype(jnp.bfloat16),
                                 attn.astype(jnp.bfloat16),
                                 (((0,), (0,)), ((), ())),
                                 preferred_element_type=jnp.float32)
    concat_ref[pl.ds(h * HD, HD), :] = out_ht

    @pl.when(h == H - 1)
    def _():
        y_ref[...] = jax.lax.dot_general(
            concat_ref[...], wo_ref[...], (((0,), (1,)), ((), ())),
            preferred_element_type=jnp.float32) + bo_ref[...]


@functools.partial(jax.jit, static_argnames=("interpret",))
def _sse_call(x4d, Wq, bq, Wk, bk, Wv, bv, part_emb, Wo, bo2d,
              interpret=False):
    grid = (H,)
    out = pl.pallas_call(
        _sse_head_kernel,
        grid=grid,
        in_specs=[
            pl.BlockSpec((S, 1, 1, HD), lambda h: (0, h, 0, 0)),
            pl.BlockSpec((1, HD, HD), lambda h: (h, 0, 0)),
            pl.BlockSpec((1, 1, HD), lambda h: (h, 0, 0)),
            pl.BlockSpec((1, HD, HD), lambda h: (h, 0, 0)),
            pl.BlockSpec((1, 1, HD), lambda h: (h, 0, 0)),
            pl.BlockSpec((1, HD, HD), lambda h: (h, 0, 0)),
            pl.BlockSpec((1, 1, HD), lambda h: (h, 0, 0)),
            pl.BlockSpec((1, P, HD), lambda h: (h, 0, 0)),
            pl.BlockSpec((D, D), lambda h: (0, 0)),
            pl.BlockSpec((1, D), lambda h: (0, 0)),
        ],
        out_specs=pl.BlockSpec((S, D), lambda h: (0, 0)),
        out_shape=jax.ShapeDtypeStruct((S, D), jnp.float32),
        scratch_shapes=[pltpu.VMEM((D, S), jnp.float32)],
        interpret=interpret,
    )(x4d, Wq, bq, Wk, bk, Wv, bv, part_emb, Wo, bo2d)
    return out


def kernel(x, Wq, bq, Wk, bk, Wv, bv, part_emb, Wo, bo, interpret=False):
    x4d = x.reshape(S, H, 1, HD)
    y = _sse_call(x4d, Wq, bq.reshape(H, 1, HD), Wk, bk.reshape(H, 1, HD),
                  Wv, bv.reshape(H, 1, HD), part_emb, Wo,
                  bo.reshape(1, D), interpret=interpret)
    return y.reshape(B, S, D)


# speedup vs baseline: 1.1798x; 1.1798x over previous
import functools

import jax
import jax.numpy as jnp
from jax import lax
from jax.experimental import pallas as pl
from jax.experimental.pallas import tpu as pltpu

B = 1
S = 2048
D = 768
H = 12
HD = D // H
P = 32
K = 2
R = 16
M = S // R
PR = P * R
NEG = -1e30


def _sse_head_kernel(x_ref, wq_ref, bq_ref, wk_ref, bk_ref, wv_ref, bv_ref,
                     pe_ref, wo_ref, bo_ref, y_ref, concat_ref):
    h = pl.program_id(0)
    xh = x_ref[:, 0, 0, :]
    wq = wq_ref[0]
    wk = wk_ref[0]
    wv = wv_ref[0]
    pe = pe_ref[0]

    q = jnp.dot(xh, wq, preferred_element_type=jnp.float32) + bq_ref[0]
    k = jnp.dot(xh, wk, preferred_element_type=jnp.float32) + bk_ref[0]
    v = jnp.dot(xh, wv, preferred_element_type=jnp.float32) + bv_ref[0]

    logits = jax.lax.dot_general(pe, q, (((1,), (1,)), ((), ())),
                                 preferred_element_type=jnp.float32)
    iota_p = lax.broadcasted_iota(jnp.int32, (P, S), 0)
    v1 = jnp.max(logits, axis=0, keepdims=True)
    i1 = jnp.min(jnp.where(logits == v1, iota_p, P), axis=0, keepdims=True)
    hit1 = iota_p == i1
    l2 = jnp.where(hit1, NEG, logits)
    v2 = jnp.max(l2, axis=0, keepdims=True)
    i2 = jnp.min(jnp.where(l2 == v2, iota_p, P), axis=0, keepdims=True)
    hit2 = iota_p == i2
    e = jnp.exp(v2 - v1)
    g1 = 1.0 / (1.0 + e)
    g2 = e / (1.0 + e)

    w_sp = jnp.where(hit1, g1, 0.0) + jnp.where(hit2, g2, 0.0)
    addm = jnp.where(hit1 | hit2, 0.0, NEG)

    eye_p = (lax.broadcasted_iota(jnp.int32, (P, P), 0) ==
             lax.broadcasted_iota(jnp.int32, (P, P), 1)).astype(jnp.float32)
    w_sp2 = jax.lax.dot_general(w_sp, eye_p, (((0,), (0,)), ((), ())),
                                preferred_element_type=jnp.float32)

    w4 = w_sp2.reshape(M, R, P)
    k3 = k.reshape(M, R, HD)
    v3 = v.reshape(M, R, HD)
    dn = (((0,), (0,)), ((1,), (1,)))
    st_k = jax.lax.dot_general(w4, k3, dn,
                               preferred_element_type=jnp.float32).reshape(PR, HD)
    st_v = jax.lax.dot_general(w4, v3, dn,
                               preferred_element_type=jnp.float32).reshape(PR, HD)

    q8 = q * (1.0 / 8.0)
    scores = jax.lax.dot_general(st_k, q8, (((1,), (1,)), ((), ())),
                                 preferred_element_type=jnp.float32)
    addm_b = jnp.broadcast_to(addm[None, :, :], (R, P, S)).reshape(PR, S)
    masked = scores + addm_b
    m = jnp.max(masked, axis=0, keepdims=True)
    ex = jnp.exp(masked - m)
    attn = ex * (1.0 / jnp.sum(ex, axis=0, keepdims=True))

    eye_pr = (lax.broadcasted_iota(jnp.int32, (PR, PR), 0) ==
              lax.broadcasted_iota(jnp.int32, (PR, PR), 1)).astype(jnp.float32)
    st_vt = jax.lax.dot_general(st_v, eye_pr, (((0,), (0,)), ((), ())),
                                preferred_element_type=jnp.float32)
    out_ht = jax.lax.dot_general(st_vt, attn, (((1,), (0,)), ((), ())),
                                 preferred_element_type=jnp.float32)
    concat_ref[pl.ds(h * HD, HD), :] = out_ht

    @pl.when(h == H - 1)
    def _():
        y_ref[...] = jax.lax.dot_general(
            concat_ref[...], wo_ref[...], (((0,), (1,)), ((), ())),
            preferred_element_type=jnp.float32) + bo_ref[...]


@functools.partial(jax.jit, static_argnames=("interpret",))
def _sse_call(x4d, Wq, bq, Wk, bk, Wv, bv, part_emb, Wo, bo2d,
              interpret=False):
    grid = (H,)
    out = pl.pallas_call(
        _sse_head_kernel,
        grid=grid,
        in_specs=[
            pl.BlockSpec((S, 1, 1, HD), lambda h: (0, h, 0, 0)),
            pl.BlockSpec((1, HD, HD), lambda h: (h, 0, 0)),
            pl.BlockSpec((1, 1, HD), lambda h: (h, 0, 0)),
            pl.BlockSpec((1, HD, HD), lambda h: (h, 0, 0)),
            pl.BlockSpec((1, 1, HD), lambda h: (h, 0, 0)),
            pl.BlockSpec((1, HD, HD), lambda h: (h, 0, 0)),
            pl.BlockSpec((1, 1, HD), lambda h: (h, 0, 0)),
            pl.BlockSpec((1, P, HD), lambda h: (h, 0, 0)),
            pl.BlockSpec((D, D), lambda h: (0, 0)),
            pl.BlockSpec((1, D), lambda h: (0, 0)),
        ],
        out_specs=pl.BlockSpec((S, D), lambda h: (0, 0)),
        out_shape=jax.ShapeDtypeStruct((S, D), jnp.float32),
        scratch_shapes=[pltpu.VMEM((D, S), jnp.float32)],
        interpret=interpret,
    )(x4d, Wq, bq, Wk, bk, Wv, bv, part_emb, Wo, bo2d)
    return out


def kernel(x, Wq, bq, Wk, bk, Wv, bv, part_emb, Wo, bo, interpret=False):
    x4d = x.reshape(S, H, 1, HD)
    y = _sse_call(x4d, Wq, bq.reshape(H, 1, HD), Wk, bk.reshape(H, 1, HD),
                  Wv, bv.reshape(H, 1, HD), part_emb, Wo,
                  bo.reshape(1, D), interpret=interpret)
    return y.reshape(B, S, D)


# speedup vs baseline: 2.2514x; 1.9082x over previous
import functools

import jax
import jax.numpy as jnp
from jax import lax
from jax.experimental import pallas as pl
from jax.experimental.pallas import tpu as pltpu

B = 1
S = 2048
D = 768
H = 12
HD = D // H
P = 32
K = 2
R = 16
M = S // R
PR = P * R
NEG = -1e30


def _sse_kernel(x_ref, wq_ref, bq_ref, wk_ref, bk_ref, wv_ref, bv_ref,
                pe_ref, wo_ref, bo_ref, y_ref, concat_ref):
    iota_p = lax.broadcasted_iota(jnp.int32, (P, S), 0)
    eye_p = (lax.broadcasted_iota(jnp.int32, (P, P), 0) ==
             lax.broadcasted_iota(jnp.int32, (P, P), 1)).astype(jnp.float32)
    eye_pr = (lax.broadcasted_iota(jnp.int32, (PR, PR), 0) ==
              lax.broadcasted_iota(jnp.int32, (PR, PR), 1)).astype(jnp.float32)

    for h in range(H):
        xh = x_ref[:, h * HD:(h + 1) * HD]
        wq = wq_ref[h]
        wk = wk_ref[h]
        wv = wv_ref[h]
        pe = pe_ref[h]

        q = jnp.dot(xh, wq, preferred_element_type=jnp.float32) + bq_ref[h:h + 1]
        k = jnp.dot(xh, wk, preferred_element_type=jnp.float32) + bk_ref[h:h + 1]
        v = jnp.dot(xh, wv, preferred_element_type=jnp.float32) + bv_ref[h:h + 1]

        logits = jax.lax.dot_general(pe, q, (((1,), (1,)), ((), ())),
                                     preferred_element_type=jnp.float32)
        v1 = jnp.max(logits, axis=0, keepdims=True)
        i1 = jnp.min(jnp.where(logits == v1, iota_p, P), axis=0, keepdims=True)
        hit1 = iota_p == i1
        l2 = jnp.where(hit1, NEG, logits)
        v2 = jnp.max(l2, axis=0, keepdims=True)
        i2 = jnp.min(jnp.where(l2 == v2, iota_p, P), axis=0, keepdims=True)
        hit2 = iota_p == i2
        e = jnp.exp(v2 - v1)
        g1 = 1.0 / (1.0 + e)
        g2 = e / (1.0 + e)

        w_sp = jnp.where(hit1, g1, 0.0) + jnp.where(hit2, g2, 0.0)
        addm = jnp.where(hit1 | hit2, 0.0, NEG)

        w_sp2 = jax.lax.dot_general(w_sp, eye_p, (((0,), (0,)), ((), ())),
                                    preferred_element_type=jnp.float32)

        w4 = w_sp2.reshape(M, R, P)
        k3 = k.reshape(M, R, HD)
        v3 = v.reshape(M, R, HD)
        dn = (((0,), (0,)), ((1,), (1,)))
        st_k = jax.lax.dot_general(
            w4, k3, dn, preferred_element_type=jnp.float32).reshape(PR, HD)
        st_v = jax.lax.dot_general(
            w4, v3, dn, preferred_element_type=jnp.float32).reshape(PR, HD)

        q8 = q * (1.0 / 8.0)
        scores = jax.lax.dot_general(st_k, q8, (((1,), (1,)), ((), ())),
                                     preferred_element_type=jnp.float32)
        addm_b = jnp.broadcast_to(addm[None, :, :], (R, P, S)).reshape(PR, S)
        masked = scores + addm_b
        m = jnp.max(masked, axis=0, keepdims=True)
        ex = jnp.exp(masked - m)
        attn = ex * (1.0 / jnp.sum(ex, axis=0, keepdims=True))

        st_vt = jax.lax.dot_general(st_v, eye_pr, (((0,), (0,)), ((), ())),
                                    preferred_element_type=jnp.float32)
        out_ht = jax.lax.dot_general(st_vt, attn, (((1,), (0,)), ((), ())),
                                     preferred_element_type=jnp.float32)
        concat_ref[h * HD:(h + 1) * HD, :] = out_ht

    y_ref[...] = jax.lax.dot_general(
        concat_ref[...], wo_ref[...], (((0,), (1,)), ((), ())),
        preferred_element_type=jnp.float32) + bo_ref[...]


@functools.partial(jax.jit, static_argnames=("interpret",))
def _sse_call(x2d, Wq, bq, Wk, bk, Wv, bv, part_emb, Wo, bo2d,
              interpret=False):
    out = pl.pallas_call(
        _sse_kernel,
        out_shape=jax.ShapeDtypeStruct((S, D), jnp.float32),
        scratch_shapes=[pltpu.VMEM((D, S), jnp.float32)],
        interpret=interpret,
    )(x2d, Wq, bq, Wk, bk, Wv, bv, part_emb, Wo, bo2d)
    return out


def kernel(x, Wq, bq, Wk, bk, Wv, bv, part_emb, Wo, bo, interpret=False):
    x2d = x.reshape(S, D)
    y = _sse_call(x2d, Wq, bq, Wk, bk, Wv, bv, part_emb, Wo,
                  bo.reshape(1, D), interpret=interpret)
    return y.reshape(B, S, D)
